# Initial kernel scaffold; baseline (speedup 1.0000x reference)
import jax
import jax.numpy as jnp
from jax.experimental import pallas as pl


def kernel(values, indices):
    raise NotImplementedError("write your pallas kernel here")


# speedup vs baseline: 1.0767x; 1.0767x over previous
import jax
import jax.numpy as jnp
from jax.experimental import pallas as pl
from jax.experimental.pallas import tpu as pltpu

_UNITS = 128
_ROWS_PER_BLOCK = 2000

_KS = (0, 42, 0x1BD11BDA ^ 42)
_ROT = ((13, 15, 26, 6), (17, 29, 16, 24))


def _dropout_body(v_ref, o_ref):
    pid = pl.program_id(0)
    shape = v_ref.shape
    base = pid * (shape[0] * shape[1])
    row = jax.lax.broadcasted_iota(jnp.int32, shape, 0)
    col = jax.lax.broadcasted_iota(jnp.int32, shape, 1)
    ctr = base + row * shape[1] + col

    ks = [jnp.int32(k) for k in _KS]

    def rotl(x, r):
        return (x << r) | jax.lax.shift_right_logical(x, 32 - r)

    def rounds(x0, x1, rots):
        for r in rots:
            x0 = x0 + x1
            x1 = rotl(x1, r) ^ x0
        return x0, x1

    x0 = jnp.full(shape, _KS[0], dtype=jnp.int32)
    x1 = ctr + ks[1]
    for i, (a, b, grp) in enumerate(
        ((1, 2, 0), (2, 0, 1), (0, 1, 0), (1, 2, 1), (2, 0, 0)), start=1):
        x0, x1 = rounds(x0, x1, _ROT[grp])
        x0 = x0 + ks[a]
        x1 = x1 + (ks[b] + jnp.int32(i))
    bits = x0 ^ x1
    o_ref[...] = jnp.where(bits >= 0, v_ref[...] * 2.0, 0.0)


def kernel(values, indices):
    del indices
    n_rows = values.shape[0] // _UNITS
    vals = values.reshape(n_rows, _UNITS)
    grid = n_rows // _ROWS_PER_BLOCK
    out = pl.pallas_call(
        _dropout_body,
        grid=(grid,),
        in_specs=[pl.BlockSpec((_ROWS_PER_BLOCK, _UNITS), lambda i: (i, 0))],
        out_specs=pl.BlockSpec((_ROWS_PER_BLOCK, _UNITS), lambda i: (i, 0)),
        out_shape=jax.ShapeDtypeStruct((n_rows, _UNITS), jnp.float32),
        compiler_params=pltpu.CompilerParams(
            dimension_semantics=("arbitrary",)),
    )(vals)
    return out.reshape(-1)


# speedup vs baseline: 1.0864x; 1.0091x over previous
import jax
import jax.numpy as jnp
from jax.experimental import pallas as pl
from jax.experimental.pallas import tpu as pltpu

_UNITS = 128
_ROWS_PER_BLOCK = 2000

_KS = (0, 42, 0x1BD11BDA ^ 42)
_ROT = ((13, 15, 26, 6), (17, 29, 16, 24))


def _dropout_body(v_ref, o_ref):
    pid = pl.program_id(0)
    shape = v_ref.shape
    base = pid * (shape[0] * shape[1])
    row = jax.lax.broadcasted_iota(jnp.int32, shape, 0)
    col = jax.lax.broadcasted_iota(jnp.int32, shape, 1)
    ctr = (base + _KS[1]) + ((row << 7) | col)

    ks = [jnp.int32(k) for k in _KS]

    def rotl(x, r):
        return (x << r) | jax.lax.shift_right_logical(x, 32 - r)

    def rounds(x0, x1, rots):
        for r in rots:
            x0 = x0 + x1
            x1 = rotl(x1, r) ^ x0
        return x0, x1

    x0 = ctr
    x1 = rotl(ctr, _ROT[0][0]) ^ ctr
    x0, x1 = rounds(x0, x1, _ROT[0][1:])
    for i, (a, b, grp) in enumerate(
        ((1, 2, 0), (2, 0, 1), (0, 1, 0), (1, 2, 1), (2, 0, 0)), start=1):
        x0 = x0 + ks[a]
        x1 = x1 + (ks[b] + jnp.int32(i))
        if i < 5:
            x0, x1 = rounds(x0, x1, _ROT[(grp + 1) % 2])
    bits = x0 ^ x1
    o_ref[...] = jnp.where(bits >= 0, v_ref[...] * 2.0, 0.0)


def kernel(values, indices):
    del indices
    n_rows = values.shape[0] // _UNITS
    vals = values.reshape(n_rows, _UNITS)
    grid = n_rows // _ROWS_PER_BLOCK
    out = pl.pallas_call(
        _dropout_body,
        grid=(grid,),
        in_specs=[pl.BlockSpec((_ROWS_PER_BLOCK, _UNITS), lambda i: (i, 0))],
        out_specs=pl.BlockSpec((_ROWS_PER_BLOCK, _UNITS), lambda i: (i, 0)),
        out_shape=jax.ShapeDtypeStruct((n_rows, _UNITS), jnp.float32),
        compiler_params=pltpu.CompilerParams(
            dimension_semantics=("parallel",)),
    )(vals)
    return out.reshape(-1)
